# baseline (speedup 1.0000x reference)
import functools

import jax
import jax.numpy as jnp
from jax import lax
from jax.experimental import pallas as pl
from jax.experimental.pallas import tpu as pltpu
from jax.experimental.pallas import tpu_sc as plsc

_V = 26
_E = 64
_BATCH = 4096
_OUT_D = _V * _E
_NP = _V // 2
_NC = 2
_NS = 16
_NW = _NC * _NS
_ROWS_W = _BATCH // _NW
_TR_W = _ROWS_W // 8
_TR_CH = 2
_NCHUNK = _TR_W // _TR_CH
_ROWS_CH = 8 * _TR_CH
_PPT = 48

_mesh = plsc.VectorSubcoreMesh(core_axis_name="c", subcore_axis_name="s")


@functools.partial(
    pl.kernel,
    mesh=_mesh,
    out_type=jax.ShapeDtypeStruct((_BATCH, _OUT_D), jnp.float32),
    scratch_types=[
        pltpu.VMEM((_ROWS_W, _V), jnp.int32),
        pltpu.VMEM((_TR_W, 128), jnp.int32),
        pltpu.VMEM((_V, _E), jnp.float32),
        pltpu.VMEM((_PPT, 2 * _E), jnp.float32),
        pltpu.VMEM((_ROWS_CH, _OUT_D), jnp.float32),
        pltpu.VMEM((_ROWS_CH, _OUT_D), jnp.float32),
        pltpu.SemaphoreType.DMA,
        pltpu.SemaphoreType.DMA,
        pltpu.SemaphoreType.DMA,
        pltpu.SemaphoreType.DMA,
        pltpu.VMEM_SHARED((_NS * _PPT, 2 * _E), jnp.float32),
    ],
    compiler_params=pltpu.CompilerParams(needs_layout_passes=False),
)
def _emb_lookup(si_hbm, table_hbm, out_hbm, raw_v, idx_v, tab26_v, pair_v,
                bw0, bw1, g0, g1, w0, w1, tab_sh):
    sid = lax.axis_index("s")
    wid = sid * _NC + lax.axis_index("c")
    row0 = wid * _ROWS_W

    pltpu.sync_copy(table_hbm, tab26_v)
    pltpu.sync_copy(si_hbm.at[wid], raw_v)
    for lp in range(_PPT):
        p = sid * _PPT + lp
        a = jnp.minimum(p // _V, _V - 1)
        b = jnp.minimum(p % _V, _V - 1)
        for q in range(4):
            pair_v[lp, pl.ds(16 * q, 16)] = tab26_v[a, pl.ds(16 * q, 16)]
            pair_v[lp, pl.ds(_E + 16 * q, 16)] = tab26_v[b, pl.ds(16 * q, 16)]
    pltpu.sync_copy(pair_v, tab_sh.at[pl.ds(sid * _PPT, _PPT)])

    iota16 = lax.iota(jnp.int32, 16)
    rowsel = iota16 >> 3
    colsel = iota16 & 7
    for t2 in range(_TR_W // 2):
        rows = t2 * 16 + iota16
        drows = t2 * 2 + rowsel
        for c in range(_NP):
            a = plsc.load_gather(raw_v, [rows, jnp.full((16,), 2 * c, jnp.int32)])
            b = plsc.load_gather(raw_v, [rows, jnp.full((16,), 2 * c + 1, jnp.int32)])
            plsc.store_scatter(idx_v, [drows, c * 8 + colsel], a * _V + b)

    plsc.subcore_barrier()

    bufw = (bw0, bw1)
    gsems = (g0, g1)
    wsems = (w0, w1)

    def gather_chunk(j):
        b = j % 2
        cps = []
        for tt in range(_TR_CH):
            for c in range(_NP):
                cps.append(pltpu.async_copy(
                    tab_sh.at[idx_v.at[j * _TR_CH + tt, pl.ds(c * 8, 8)]],
                    bufw[b].at[pl.ds(tt * 8, 8), pl.ds(c * 128, 128)],
                    gsems[b],
                ))
        return cps

    gathers = [None] * _NCHUNK
    writes = [None] * _NCHUNK
    gathers[0] = gather_chunk(0)
    for j in range(_NCHUNK):
        b = j % 2
        for cp in gathers[j]:
            cp.wait()
        writes[j] = pltpu.async_copy(
            bufw[b],
            out_hbm.at[pl.ds(row0 + j * _ROWS_CH, _ROWS_CH)],
            wsems[b],
        )
        if j + 1 < _NCHUNK:
            if j >= 1:
                writes[j - 1].wait()
            gathers[j + 1] = gather_chunk(j + 1)
    writes[_NCHUNK - 2].wait()
    writes[_NCHUNK - 1].wait()


def kernel(static_input, table):
    si3 = static_input.astype(jnp.int32).reshape(_NW, _ROWS_W, _V)
    return _emb_lookup(si3, table.astype(jnp.float32))


# speedup vs baseline: 1.0171x; 1.0171x over previous
import functools

import jax
import jax.numpy as jnp
from jax import lax
from jax.experimental import pallas as pl
from jax.experimental.pallas import tpu as pltpu
from jax.experimental.pallas import tpu_sc as plsc

_V = 26
_E = 64
_BATCH = 4096
_OUT_D = _V * _E
_NP = _V // 2
_NC = 2
_NS = 16
_NW = _NC * _NS
_ROWS_W = _BATCH // _NW
_TR_W = _ROWS_W // 8
_TR_CH = 2
_NCHUNK = _TR_W // _TR_CH
_ROWS_CH = 8 * _TR_CH
_PPT = 48

_mesh = plsc.VectorSubcoreMesh(core_axis_name="c", subcore_axis_name="s")


@functools.partial(
    pl.kernel,
    mesh=_mesh,
    out_type=jax.ShapeDtypeStruct((_BATCH, _OUT_D), jnp.float32),
    scratch_types=[
        pltpu.VMEM((_ROWS_W, _V), jnp.int32),
        pltpu.VMEM((_TR_W, 128), jnp.int32),
        pltpu.VMEM((_V, _E), jnp.float32),
        pltpu.VMEM((_PPT, 2 * _E), jnp.float32),
        pltpu.VMEM((_ROWS_CH, _OUT_D), jnp.float32),
        pltpu.VMEM((_ROWS_CH, _OUT_D), jnp.float32),
        pltpu.SemaphoreType.DMA,
        pltpu.SemaphoreType.DMA,
        pltpu.SemaphoreType.DMA,
        pltpu.SemaphoreType.DMA,
        pltpu.SemaphoreType.DMA,
        pltpu.SemaphoreType.DMA,
        pltpu.SemaphoreType.DMA,
        pltpu.VMEM_SHARED((_NS * _PPT, 2 * _E), jnp.float32),
    ],
    compiler_params=pltpu.CompilerParams(needs_layout_passes=False),
)
def _emb_lookup(si_hbm, table_hbm, out_hbm, raw_v, idx_v, tab26_v, pair_v,
                bw0, bw1, g0, g1, w0, w1, s_tab, s_si, s_pw, tab_sh):
    sid = lax.axis_index("s")
    wid = sid * _NC + lax.axis_index("c")
    row0 = wid * _ROWS_W

    cp_tab = pltpu.async_copy(table_hbm, tab26_v, s_tab)
    cp_si = pltpu.async_copy(si_hbm.at[pl.ds(row0, _ROWS_W)], raw_v, s_si)

    cp_tab.wait()
    for lp in range(_PPT):
        p = sid * _PPT + lp
        a = jnp.minimum(p // _V, _V - 1)
        b = jnp.minimum(p % _V, _V - 1)
        for q in range(4):
            pair_v[lp, pl.ds(16 * q, 16)] = tab26_v[a, pl.ds(16 * q, 16)]
            pair_v[lp, pl.ds(_E + 16 * q, 16)] = tab26_v[b, pl.ds(16 * q, 16)]
    cp_pw = pltpu.async_copy(pair_v, tab_sh.at[pl.ds(sid * _PPT, _PPT)], s_pw)

    cp_si.wait()
    iota16 = lax.iota(jnp.int32, 16)
    rowsel = iota16 >> 3
    colsel = iota16 & 7
    for t2 in range(_TR_W // 2):
        rows = t2 * 16 + iota16
        drows = t2 * 2 + rowsel
        for c in range(_NP):
            a = plsc.load_gather(raw_v, [rows, jnp.full((16,), 2 * c, jnp.int32)])
            b = plsc.load_gather(raw_v, [rows, jnp.full((16,), 2 * c + 1, jnp.int32)])
            plsc.store_scatter(idx_v, [drows, c * 8 + colsel], a * _V + b)

    cp_pw.wait()
    plsc.subcore_barrier()

    bufw = (bw0, bw1)
    gsems = (g0, g1)
    wsems = (w0, w1)

    def gather_chunk(j):
        b = j % 2
        cps = []
        for tt in range(_TR_CH):
            for c in range(_NP):
                cps.append(pltpu.async_copy(
                    tab_sh.at[idx_v.at[j * _TR_CH + tt, pl.ds(c * 8, 8)]],
                    bufw[b].at[pl.ds(tt * 8, 8), pl.ds(c * 128, 128)],
                    gsems[b],
                ))
        return cps

    gathers = [None] * _NCHUNK
    writes = [None] * _NCHUNK
    gathers[0] = gather_chunk(0)
    for j in range(_NCHUNK):
        b = j % 2
        for cp in gathers[j]:
            cp.wait()
        writes[j] = pltpu.async_copy(
            bufw[b],
            out_hbm.at[pl.ds(row0 + j * _ROWS_CH, _ROWS_CH)],
            wsems[b],
        )
        if j + 1 < _NCHUNK:
            if j >= 1:
                writes[j - 1].wait()
            gathers[j + 1] = gather_chunk(j + 1)
    writes[_NCHUNK - 2].wait()
    writes[_NCHUNK - 1].wait()


def kernel(static_input, table):
    return _emb_lookup(static_input.astype(jnp.int32), table.astype(jnp.float32))


# speedup vs baseline: 1.0649x; 1.0470x over previous
import functools

import jax
import jax.numpy as jnp
from jax import lax
from jax.experimental import pallas as pl
from jax.experimental.pallas import tpu as pltpu
from jax.experimental.pallas import tpu_sc as plsc

_V = 26
_E = 64
_BATCH = 4096
_OUT_D = _V * _E
_NP = _V // 2
_NC = 2
_NS = 16
_NW = _NC * _NS
_ROWS_W = _BATCH // _NW
_TR_W = _ROWS_W // 8
_TR_CH = 2
_NCHUNK = _TR_W // _TR_CH
_ROWS_CH = 8 * _TR_CH

_mesh = plsc.VectorSubcoreMesh(core_axis_name="c", subcore_axis_name="s")


@functools.partial(
    pl.kernel,
    mesh=_mesh,
    out_type=jax.ShapeDtypeStruct((_BATCH, _OUT_D), jnp.float32),
    scratch_types=[
        pltpu.VMEM((_ROWS_W, _V), jnp.int32),
        pltpu.VMEM((_TR_W, 128), jnp.int32),
        pltpu.VMEM((_ROWS_CH, _OUT_D), jnp.float32),
        pltpu.VMEM((_ROWS_CH, _OUT_D), jnp.float32),
        pltpu.SemaphoreType.DMA,
        pltpu.SemaphoreType.DMA,
        pltpu.SemaphoreType.DMA,
        pltpu.SemaphoreType.DMA,
        pltpu.SemaphoreType.DMA,
        pltpu.SemaphoreType.DMA,
        pltpu.VMEM_SHARED((_V * _V, 128), jnp.float32),
    ],
    compiler_params=pltpu.CompilerParams(needs_layout_passes=False),
)
def _emb_lookup(si_hbm, tab2_hbm, out_hbm, raw_v, idx_v,
                bw0, bw1, g0, g1, w0, w1, s_t2, s_si, tab_sh):
    sid = lax.axis_index("s")
    wid = sid * _NC + lax.axis_index("c")
    row0 = wid * _ROWS_W

    @pl.when(sid == 0)
    def _():
        pltpu.async_copy(tab2_hbm, tab_sh, s_t2)
    cp_si = pltpu.async_copy(si_hbm.at[pl.ds(row0, _ROWS_W)], raw_v, s_si)

    cp_si.wait()
    iota16 = lax.iota(jnp.int32, 16)
    rowsel = iota16 >> 3
    colsel = iota16 & 7
    for t2 in range(_TR_W // 2):
        rows = t2 * 16 + iota16
        drows = t2 * 2 + rowsel
        for c in range(_NP):
            a = plsc.load_gather(raw_v, [rows, jnp.full((16,), 2 * c, jnp.int32)])
            b = plsc.load_gather(raw_v, [rows, jnp.full((16,), 2 * c + 1, jnp.int32)])
            plsc.store_scatter(idx_v, [drows, c * 8 + colsel], a * _V + b)

    @pl.when(sid == 0)
    def _():
        pltpu.make_async_copy(tab2_hbm, tab_sh, s_t2).wait()
    plsc.subcore_barrier()

    bufw = (bw0, bw1)
    gsems = (g0, g1)
    wsems = (w0, w1)

    def gather_chunk(j):
        b = j % 2
        cps = []
        for tt in range(_TR_CH):
            for c in range(_NP):
                cps.append(pltpu.async_copy(
                    tab_sh.at[idx_v.at[j * _TR_CH + tt, pl.ds(c * 8, 8)]],
                    bufw[b].at[pl.ds(tt * 8, 8), pl.ds(c * 128, 128)],
                    gsems[b],
                ))
        return cps

    gathers = [None] * _NCHUNK
    writes = [None] * _NCHUNK
    gathers[0] = gather_chunk(0)
    for j in range(_NCHUNK):
        b = j % 2
        for cp in gathers[j]:
            cp.wait()
        writes[j] = pltpu.async_copy(
            bufw[b],
            out_hbm.at[pl.ds(row0 + j * _ROWS_CH, _ROWS_CH)],
            wsems[b],
        )
        if j + 1 < _NCHUNK:
            if j >= 1:
                writes[j - 1].wait()
            gathers[j + 1] = gather_chunk(j + 1)
    writes[_NCHUNK - 2].wait()
    writes[_NCHUNK - 1].wait()


def kernel(static_input, table):
    tab2 = jnp.concatenate(
        [
            jnp.broadcast_to(table[:, None, :], (_V, _V, _E)),
            jnp.broadcast_to(table[None, :, :], (_V, _V, _E)),
        ],
        axis=-1,
    ).reshape(_V * _V, 2 * _E)
    return _emb_lookup(static_input.astype(jnp.int32), tab2)
